# baseline (speedup 1.0000x reference)
import functools

import jax
import jax.numpy as jnp
from jax import lax
from jax.experimental import pallas as pl
from jax.experimental.pallas import tpu as pltpu
from jax.experimental.pallas import tpu_sc as plsc

N = 10000
E = 320000
NC = 2
NS = 16
NW = NC * NS
NP = 10112
RPS = NP // NS


def _tc_in_body(x_ref, wm_ref, wd_ref, om_ref, od_ref):
    x = x_ref[...]
    for c in range(NC):
        om_ref[c] = jnp.dot(x, wm_ref[c], preferred_element_type=jnp.float32)
    od_ref[...] = jnp.dot(x, wd_ref[...], preferred_element_type=jnp.float32)


def _tc_in(x, w_main, w_ad):
    return pl.pallas_call(
        _tc_in_body,
        out_shape=[
            jax.ShapeDtypeStruct((NC, N, w_main.shape[2]), jnp.float32),
            jax.ShapeDtypeStruct((N, 16), jnp.float32),
        ],
    )(x, w_main, w_ad)


def _tc_mid_body(fc, next_split, acc_ref, r_ref, b_ref, wm_ref, wd_ref,
                 om_ref, od_ref):
    num = jnp.concatenate([acc_ref[0, :N, :fc], acc_ref[1, :N, :fc]], axis=1)
    den = jnp.concatenate(
        [jnp.dot(acc_ref[0, :N, fc:fc + 16], r_ref[...],
                 preferred_element_type=jnp.float32),
         jnp.dot(acc_ref[1, :N, fc:fc + 16], r_ref[...],
                 preferred_element_type=jnp.float32)], axis=1)
    h = num / (den + 1e-16) + b_ref[...]
    h = jnp.maximum(h, 0.0)
    if next_split:
        for c in range(NC):
            om_ref[c] = jnp.dot(h, wm_ref[c],
                                preferred_element_type=jnp.float32)
    else:
        om_ref[...] = jnp.dot(h, wm_ref[...],
                              preferred_element_type=jnp.float32)
    od_ref[...] = jnp.dot(h, wd_ref[...], preferred_element_type=jnp.float32)


def _tc_mid(acc, r_mat, b, w_main, w_ad, fc, next_split):
    if next_split:
        om_shape = jax.ShapeDtypeStruct((NC, N, w_main.shape[2]), jnp.float32)
    else:
        om_shape = jax.ShapeDtypeStruct((N, w_main.shape[1]), jnp.float32)
    return pl.pallas_call(
        functools.partial(_tc_mid_body, fc, next_split),
        out_shape=[om_shape, jax.ShapeDtypeStruct((N, 16), jnp.float32)],
    )(acc, r_mat, b[None, :], w_main, w_ad)


def _tc_out_body(acc_ref, b_ref, o_ref):
    a = acc_ref[0, :N] + acc_ref[1, :N]
    o_ref[...] = a[:, 0:1] / (a[:, 16:17] + 1e-16) + b_ref[...]


def _tc_out(acc, b3):
    return pl.pallas_call(
        _tc_out_body,
        out_shape=jax.ShapeDtypeStruct((N, 1), jnp.float32),
    )(acc, b3[None, :])


def _lane_gather(x, idx):
    dnums = lax.GatherDimensionNumbers(
        offset_dims=(), collapsed_slice_dims=(0,), start_index_map=(0,))
    return lax.gather(x, idx[:, None], dnums, slice_sizes=(1,),
                      mode=lax.GatherScatterMode.PROMISE_IN_BOUNDS)


@functools.lru_cache(maxsize=None)
def _make_sc_edge(fp, c_log2, g, split):
    t = fp + 16
    w_out = fp + 16
    epw = E // NS if split else E // NW
    nchunk = epw // g
    assert nchunk % 2 == 0 and g % 8 == 0
    npairs = nchunk // 2
    mesh = plsc.VectorSubcoreMesh(core_axis_name="c", subcore_axis_name="s")

    @functools.partial(
        pl.kernel,
        mesh=mesh,
        compiler_params=pltpu.CompilerParams(use_tc_tiling_on_sc=False),
        out_type=jax.ShapeDtypeStruct((NC, NP, w_out), jnp.float32),
        scratch_types=[
            pltpu.VMEM((nchunk, g), jnp.int32),
            pltpu.VMEM((nchunk, g), jnp.int32),
            pltpu.VMEM((g, t), jnp.float32),
            pltpu.VMEM((g, t), jnp.float32),
            pltpu.VMEM((g, 16), jnp.float32),
            pltpu.VMEM((g, 16), jnp.float32),
            pltpu.VMEM((g, w_out), jnp.float32),
            pltpu.VMEM((g, w_out), jnp.float32),
            pltpu.VMEM_SHARED((NP, w_out), jnp.float32),
            pltpu.SemaphoreType.DMA,
            pltpu.SemaphoreType.DMA,
            pltpu.SemaphoreType.DMA,
            pltpu.SemaphoreType.DMA,
            pltpu.SemaphoreType.DMA,
            pltpu.SemaphoreType.DMA,
        ],
    )
    def sc_edge(src_hbm, dst_hbm, hs_hbm, ad_hbm, z_hbm, out_hbm,
                isa, ida, hs0, hs1, ad0, ad1, msg0, msg1, acc_sh,
                sh0, sh1, sa0, sa1, sm0, sm1):
        cid = lax.axis_index("c")
        sid = lax.axis_index("s")
        pltpu.sync_copy(z_hbm, acc_sh.at[pl.ds(sid * RPS, RPS)])
        if split:
            pltpu.sync_copy(src_hbm.at[cid, sid], isa)
            pltpu.sync_copy(dst_hbm.at[sid], ida)
        else:
            w = sid * NC + cid
            pltpu.sync_copy(src_hbm.at[w], isa)
            pltpu.sync_copy(dst_hbm.at[w], ida)
        plsc.subcore_barrier()

        bufs = ((hs0, ad0, msg0, sh0, sa0, sm0),
                (hs1, ad1, msg1, sh1, sa1, sm1))

        def issue(b, gi):
            h_v, a_v, m_v, s_h, s_a, s_m = bufs[b]
            pltpu.async_copy(hs_hbm.at[isa.at[gi]], h_v, s_h)
            pltpu.async_copy(ad_hbm.at[ida.at[gi]], a_v, s_a)

        def wait(b, gi):
            h_v, a_v, m_v, s_h, s_a, s_m = bufs[b]
            pltpu.make_async_copy(hs_hbm.at[isa.at[gi]], h_v, s_h).wait()
            pltpu.make_async_copy(ad_hbm.at[ida.at[gi]], a_v, s_a).wait()

        def wait_scatter(b, gi):
            h_v, a_v, m_v, s_h, s_a, s_m = bufs[b]
            pltpu.make_async_copy(m_v, acc_sh.at[ida.at[gi]], s_m).wait()

        def compute_scatter(b, gi, p):
            h_v, a_v, m_v, s_h, s_a, s_m = bufs[b]

            @pl.when(p > 0)
            def _():
                wait_scatter(b, gi)

            @plsc.parallel_loop(0, g, 1, unroll=4)
            def edge_body(e):
                asv = h_v[e, pl.ds(fp, 16)]
                adv = a_v[e, :]
                if split:
                    sel = jnp.bitwise_and(
                        lax.iota(jnp.int32, 16) + 4 * cid, 15)
                    adv = _lane_gather(adv, sel)
                z = asv + adv
                z = jnp.where(z >= 0.0, z, 0.2 * z)
                wv = jnp.exp(z)
                msg_ref = m_v
                msg_ref[e, pl.ds(fp, 16)] = wv
                for k in range(fp // 16):
                    if c_log2 == 0:
                        wk = wv
                    else:
                        idxk = lax.shift_right_logical(
                            lax.iota(jnp.int32, 16) + 16 * k, c_log2)
                        wk = _lane_gather(wv, idxk)
                    msg_ref[e, pl.ds(16 * k, 16)] = (
                        h_v[e, pl.ds(16 * k, 16)] * wk)

            pltpu.async_copy(m_v, acc_sh.at[ida.at[gi]], s_m, add=True)

        issue(0, 0)

        def pair_body(p, carry):
            gi = 2 * p
            wait(0, gi)
            issue(1, gi + 1)
            compute_scatter(0, gi, p)
            wait(1, gi + 1)

            @pl.when(p < npairs - 1)
            def _():
                issue(0, gi + 2)

            compute_scatter(1, gi + 1, p)
            return carry

        lax.fori_loop(0, npairs, pair_body, 0)
        wait_scatter(0, nchunk - 2)
        wait_scatter(1, nchunk - 1)
        plsc.subcore_barrier()
        pltpu.sync_copy(acc_sh.at[pl.ds(sid * RPS, RPS)],
                        out_hbm.at[cid, pl.ds(sid * RPS, RPS)])

    return sc_edge


def _fold(w, a_src, a_dst, fp, split):
    d, f = w.shape
    h, c = a_src.shape
    rows = jnp.arange(f)
    s_dst = jnp.zeros((f, 16), jnp.float32).at[rows, rows // c].set(
        a_dst.reshape(-1))
    w_ad = w @ s_dst
    if split:
        fc = f // 2
        parts = []
        for core in range(2):
            wc = w[:, core * fc:(core + 1) * fc]
            rc = jnp.arange(fc)
            sc = jnp.zeros((fc, 16), jnp.float32).at[rc, rc // c].set(
                a_src[core * (h // 2):(core + 1) * (h // 2)].reshape(-1))
            parts.append(jnp.concatenate([wc, wc @ sc], axis=1))
        return jnp.stack(parts), w_ad
    s_src = jnp.zeros((f, 16), jnp.float32).at[rows, rows // c].set(
        a_src.reshape(-1))
    w_pad = jnp.pad(w, ((0, 0), (0, fp - f)))
    return jnp.concatenate([w_pad, w @ s_src], axis=1), w_ad


def _rmat(c, f):
    cols = jnp.arange(f)
    return (jnp.arange(16)[:, None] == (cols[None, :] // c)).astype(
        jnp.float32)


def kernel(x, edge_index, W1, a_src1, a_dst1, b1, W2, a_src2, a_dst2, b2,
           W3, a_src3, a_dst3, b3):
    src = edge_index[0].astype(jnp.int32)
    dst = edge_index[1].astype(jnp.int32)

    wm1, wd1 = _fold(W1, a_src1, a_dst1, 128, True)
    wm2, wd2 = _fold(W2, a_src2, a_dst2, 64, True)
    wm3, wd3 = _fold(W3, a_src3, a_dst3, 16, False)

    z1 = jnp.zeros((RPS, 64 + 16), jnp.float32)
    z2 = jnp.zeros((RPS, 32 + 16), jnp.float32)
    z3 = jnp.zeros((RPS, 16 + 16), jnp.float32)

    g1, g2, g3 = 80, 80, 200
    sc_l1 = _make_sc_edge(64, 4, g1, True)
    sc_l2 = _make_sc_edge(32, 3, g2, True)
    sc_l3 = _make_sc_edge(16, 0, g3, False)

    def chunked_split(a, g):
        a2 = a.reshape(NS, (E // NS) // g, g)
        return jnp.stack([a2, a2 + N])

    def chunked(a, g):
        return a.reshape(NW, (E // NW) // g, g)

    hs1, ad1t = _tc_in(x, wm1, wd1)
    acc1 = sc_l1(chunked_split(src, g1), chunked(dst, g1).reshape(
        NS, NC * ((E // NW) // g1), g1), hs1.reshape(NC * N, 80), ad1t, z1)
    hs2, ad2t = _tc_mid(acc1, _rmat(16, 64), b1, wm2, wd2, 64, True)
    acc2 = sc_l2(chunked_split(src, g2), chunked(dst, g2).reshape(
        NS, NC * ((E // NW) // g2), g2), hs2.reshape(NC * N, 48), ad2t, z2)
    hs3, ad3t = _tc_mid(acc2, _rmat(8, 32), b2, wm3, wd3, 32, False)
    acc3 = sc_l3(chunked(src, g3), chunked(dst, g3), hs3, ad3t, z3)
    return _tc_out(acc3, b3)


# speedup vs baseline: 1.1957x; 1.1957x over previous
import functools

import jax
import jax.numpy as jnp
from jax import lax
from jax.experimental import pallas as pl
from jax.experimental.pallas import tpu as pltpu
from jax.experimental.pallas import tpu_sc as plsc

N = 10000
E = 320000
NC = 2
NS = 16
NW = NC * NS
EPW = E // NW
NP = 10112
RPS = NP // NS


def _tc_in_body(x_ref, wm_ref, wd_ref, om_ref, od_ref):
    x = x_ref[...]
    om_ref[...] = jnp.dot(x, wm_ref[...], preferred_element_type=jnp.float32)
    od_ref[...] = jnp.dot(x, wd_ref[...], preferred_element_type=jnp.float32)


def _tc_in(x, w_main, w_ad):
    return pl.pallas_call(
        _tc_in_body,
        out_shape=[
            jax.ShapeDtypeStruct((N, w_main.shape[1]), jnp.float32),
            jax.ShapeDtypeStruct((N, 16), jnp.float32),
        ],
    )(x, w_main, w_ad)


def _tc_mid_body(fp, acc_ref, r_ref, b_ref, wm_ref, wd_ref, om_ref, od_ref):
    a = acc_ref[0, :N] + acc_ref[1, :N]
    num = a[:, :fp]
    sv = a[:, fp:fp + 16]
    den = jnp.dot(sv, r_ref[...], preferred_element_type=jnp.float32)
    h = num / (den + 1e-16) + b_ref[...]
    h = jnp.maximum(h, 0.0)
    om_ref[...] = jnp.dot(h, wm_ref[...], preferred_element_type=jnp.float32)
    od_ref[...] = jnp.dot(h, wd_ref[...], preferred_element_type=jnp.float32)


def _tc_mid(acc, r_mat, b, w_main, w_ad, fp):
    return pl.pallas_call(
        functools.partial(_tc_mid_body, fp),
        out_shape=[
            jax.ShapeDtypeStruct((N, w_main.shape[1]), jnp.float32),
            jax.ShapeDtypeStruct((N, 16), jnp.float32),
        ],
    )(acc, r_mat, b[None, :], w_main, w_ad)


def _tc_out_body(acc_ref, b_ref, o_ref):
    a = acc_ref[0, :N] + acc_ref[1, :N]
    o_ref[...] = a[:, 0:1] / (a[:, 16:17] + 1e-16) + b_ref[...]


def _tc_out(acc, b3):
    return pl.pallas_call(
        _tc_out_body,
        out_shape=jax.ShapeDtypeStruct((N, 1), jnp.float32),
    )(acc, b3[None, :])


def _lane_gather(x, idx):
    dnums = lax.GatherDimensionNumbers(
        offset_dims=(), collapsed_slice_dims=(0,), start_index_map=(0,))
    return lax.gather(x, idx[:, None], dnums, slice_sizes=(1,),
                      mode=lax.GatherScatterMode.PROMISE_IN_BOUNDS)


@functools.lru_cache(maxsize=None)
def _make_sc_edge(fp, c_log2, g, amsg):
    t = fp + 16
    w_out = fp + 16
    nchunk = EPW // g
    assert nchunk % 2 == 0
    npairs = nchunk // 2
    mesh = plsc.VectorSubcoreMesh(core_axis_name="c", subcore_axis_name="s")

    scratch = [
        pltpu.VMEM((nchunk, g), jnp.int32),
        pltpu.VMEM((nchunk, g), jnp.int32),
        pltpu.VMEM((g, t), jnp.float32),
        pltpu.VMEM((g, t), jnp.float32),
        pltpu.VMEM((g, 16), jnp.float32),
        pltpu.VMEM((g, 16), jnp.float32),
    ]
    scratch += [pltpu.VMEM((g, w_out), jnp.float32)] * (2 if amsg else 1)
    scratch += [pltpu.VMEM_SHARED((NP, w_out), jnp.float32)]
    scratch += [pltpu.SemaphoreType.DMA] * (6 if amsg else 4)

    @functools.partial(
        pl.kernel,
        mesh=mesh,
        compiler_params=pltpu.CompilerParams(use_tc_tiling_on_sc=False),
        out_type=jax.ShapeDtypeStruct((NC, NP, w_out), jnp.float32),
        scratch_types=scratch,
    )
    def sc_edge(src_hbm, dst_hbm, hs_hbm, ad_hbm, z_hbm, out_hbm, *sc):
        if amsg:
            (isa, ida, hs0, hs1, ad0, ad1, msg0, msg1, acc_sh,
             sh0, sh1, sa0, sa1, sm0, sm1) = sc
            msgs, sms = (msg0, msg1), (sm0, sm1)
        else:
            (isa, ida, hs0, hs1, ad0, ad1, msg0, acc_sh,
             sh0, sh1, sa0, sa1) = sc
            msgs, sms = (msg0, msg0), (None, None)
        cid = lax.axis_index("c")
        sid = lax.axis_index("s")
        w = sid * NC + cid
        pltpu.sync_copy(z_hbm, acc_sh.at[pl.ds(sid * RPS, RPS)])
        pltpu.sync_copy(src_hbm.at[w], isa)
        pltpu.sync_copy(dst_hbm.at[w], ida)
        plsc.subcore_barrier()

        bufs = ((hs0, ad0, sh0, sa0), (hs1, ad1, sh1, sa1))

        def issue(b, gi):
            h_v, a_v, s_h, s_a = bufs[b]
            pltpu.async_copy(hs_hbm.at[isa.at[gi]], h_v, s_h)
            pltpu.async_copy(ad_hbm.at[ida.at[gi]], a_v, s_a)

        def wait(b, gi):
            h_v, a_v, s_h, s_a = bufs[b]
            pltpu.make_async_copy(hs_hbm.at[isa.at[gi]], h_v, s_h).wait()
            pltpu.make_async_copy(ad_hbm.at[ida.at[gi]], a_v, s_a).wait()

        def compute_scatter(b, gi, p):
            h_v, a_v, s_h, s_a = bufs[b]
            msg_v = msgs[b]

            if amsg:
                @pl.when(p > 0)
                def _():
                    pltpu.make_async_copy(
                        msg_v, acc_sh.at[ida.at[gi]], sms[b]).wait()

            @plsc.parallel_loop(0, g, 1, unroll=8)
            def edge_body(e):
                asv = h_v[e, pl.ds(fp, 16)]
                adv = a_v[e, :]
                z = asv + adv
                z = jnp.where(z >= 0.0, z, 0.2 * z)
                wv = jnp.exp(z)
                msg_v[e, pl.ds(fp, 16)] = wv
                for k in range(fp // 16):
                    if c_log2 == 0:
                        wk = wv
                    else:
                        idxk = lax.shift_right_logical(
                            lax.iota(jnp.int32, 16) + 16 * k, c_log2)
                        wk = _lane_gather(wv, idxk)
                    msg_v[e, pl.ds(16 * k, 16)] = (
                        h_v[e, pl.ds(16 * k, 16)] * wk)

            if amsg:
                pltpu.async_copy(msg_v, acc_sh.at[ida.at[gi]], sms[b],
                                 add=True)
            else:
                pltpu.sync_copy(msg_v, acc_sh.at[ida.at[gi]], add=True)

        issue(0, 0)

        def pair_body(p, carry):
            gi = 2 * p
            wait(0, gi)
            issue(1, gi + 1)
            compute_scatter(0, gi, p)
            wait(1, gi + 1)

            @pl.when(p < npairs - 1)
            def _():
                issue(0, gi + 2)

            compute_scatter(1, gi + 1, p)
            return carry

        lax.fori_loop(0, npairs, pair_body, 0)
        if amsg:
            pltpu.make_async_copy(
                msgs[0], acc_sh.at[ida.at[nchunk - 2]], sms[0]).wait()
            pltpu.make_async_copy(
                msgs[1], acc_sh.at[ida.at[nchunk - 1]], sms[1]).wait()
        plsc.subcore_barrier()
        pltpu.sync_copy(acc_sh.at[pl.ds(sid * RPS, RPS)],
                        out_hbm.at[cid, pl.ds(sid * RPS, RPS)])

    return sc_edge


def _fold(w, a_src, a_dst, fp):
    d, f = w.shape
    h, c = a_src.shape
    rows = jnp.arange(f)
    s_src = jnp.zeros((f, 16), jnp.float32).at[rows, rows // c].set(
        a_src.reshape(-1))
    s_dst = jnp.zeros((f, 16), jnp.float32).at[rows, rows // c].set(
        a_dst.reshape(-1))
    w_pad = jnp.pad(w, ((0, 0), (0, fp - f)))
    return jnp.concatenate([w_pad, w @ s_src], axis=1), w @ s_dst


def _rmat(c, f):
    cols = jnp.arange(f)
    return (jnp.arange(16)[:, None] == (cols[None, :] // c)).astype(
        jnp.float32)


def kernel(x, edge_index, W1, a_src1, a_dst1, b1, W2, a_src2, a_dst2, b2,
           W3, a_src3, a_dst3, b3):
    src = edge_index[0].astype(jnp.int32)
    dst = edge_index[1].astype(jnp.int32)

    wm1, wd1 = _fold(W1, a_src1, a_dst1, 128)
    wm2, wd2 = _fold(W2, a_src2, a_dst2, 64)
    wm3, wd3 = _fold(W3, a_src3, a_dst3, 16)
    r1 = _rmat(16, 128)
    r2 = _rmat(8, 64)

    z1 = jnp.zeros((RPS, 128 + 16), jnp.float32)
    z2 = jnp.zeros((RPS, 64 + 16), jnp.float32)
    z3 = jnp.zeros((RPS, 16 + 16), jnp.float32)

    g1, g2, g3 = 40, 100, 200
    sc_l1 = _make_sc_edge(128, 4, g1, False)
    sc_l2 = _make_sc_edge(64, 3, g2, True)
    sc_l3 = _make_sc_edge(16, 0, g3, True)

    def chunked(a, g):
        return a.reshape(NW, EPW // g, g)

    hs1, ad1t = _tc_in(x, wm1, wd1)
    acc1 = sc_l1(chunked(src, g1), chunked(dst, g1), hs1, ad1t, z1)
    hs2, ad2t = _tc_mid(acc1, r1, b1, wm2, wd2, 128)
    acc2 = sc_l2(chunked(src, g2), chunked(dst, g2), hs2, ad2t, z2)
    hs3, ad3t = _tc_mid(acc2, r2, b2, wm3, wd3, 64)
    acc3 = sc_l3(chunked(src, g3), chunked(dst, g3), hs3, ad3t, z3)
    return _tc_out(acc3, b3)
